# Initial kernel scaffold; baseline (speedup 1.0000x reference)
import jax
import jax.numpy as jnp
from jax.experimental import pallas as pl


def kernel(x_drug, x_protein, edge_index_dd, edge_index_dp, edge_index_pp, proj_W, proj_b, ln_g, ln_b, W_rel, W_root, conv_b, bn_g, bn_b):
    raise NotImplementedError("write your pallas kernel here")


# speedup vs baseline: 1.5218x; 1.5218x over previous
import functools

import jax
import jax.numpy as jnp
from jax import lax
from jax.experimental import pallas as pl
from jax.experimental.pallas import tpu as pltpu
from jax.experimental.pallas import tpu_sc as plsc

N = 10000
E = 160000
D = 128
H = 192
R = 3
L = 3
TN = 2 * N

NSUB = 16
HALF = TN // 2
PAD_ROWS = 10016
GARBAGE = HALF
EPS = E // NSUB
K = 80
NCH = EPS // K
ZCH = PAD_ROWS // NSUB
OCH = HALF // NSUB

_mesh = plsc.VectorSubcoreMesh(core_axis_name="c", subcore_axis_name="s")


def _compute_local_dst(dst_v, loc_v, base):
    def ch_body(ch, _):
        def j_body(j, _):
            v = dst_v[pl.ds(ch * K + j * 16, 16)] - base
            oob = (v < 0) | (v >= HALF)
            loc_v[ch, 0, pl.ds(j * 16, 16)] = jnp.where(oob, GARBAGE, v)
            return 0
        return lax.fori_loop(0, K // 16, j_body, 0)
    lax.fori_loop(0, NCH, ch_body, 0)


HH = H // 2


@functools.partial(
    pl.kernel,
    mesh=_mesh,
    compiler_params=pltpu.CompilerParams(use_tc_tiling_on_sc=False),
    out_type=jax.ShapeDtypeStruct((2, TN, HH), jnp.float32),
    scratch_types=[
        pltpu.VMEM((EPS,), jnp.int32),
        pltpu.VMEM((EPS,), jnp.int32),
        pltpu.VMEM((NCH, 1, K), jnp.int32),
        pltpu.VMEM((K, HH), jnp.float32),
        pltpu.VMEM((K, HH), jnp.float32),
        pltpu.VMEM_SHARED((PAD_ROWS, HH), jnp.float32),
        pltpu.SemaphoreType.DMA,
    ],
)
def _sc_agg(hw2, dst, src0, src1, zeros, out, dst_v, src_v, loc_v,
            rows_a, rows_b, acc, gsem):
    c = lax.axis_index("c")
    s = lax.axis_index("s")
    base = c * HALF
    e0 = s * EPS

    pltpu.sync_copy(dst.at[pl.ds(e0, EPS)], dst_v)
    _compute_local_dst(dst_v, loc_v, base)

    for p, src in ((0, src0), (1, src1)):
        pltpu.sync_copy(zeros.at[pl.ds(s * ZCH, ZCH)],
                        acc.at[pl.ds(s * ZCH, ZCH)])
        pltpu.sync_copy(src.at[pl.ds(e0, EPS)], src_v)
        plsc.subcore_barrier()

        pltpu.async_copy(hw2.at[src_v.at[pl.ds(0, K)]], rows_a, gsem)

        def body(t, _):
            pltpu.make_async_copy(hw2.at[src_v.at[pl.ds(0, K)]],
                                  rows_a, gsem).wait()
            even = (t % 2) == 0

            @pl.when(even)
            def _():
                @pl.when(t + 1 < NCH)
                def _():
                    pltpu.async_copy(hw2.at[src_v.at[pl.ds((t + 1) * K, K)]],
                                     rows_b, gsem)
                pltpu.sync_copy(rows_a, acc.at[loc_v.at[t, 0]], add=True)

            @pl.when(jnp.logical_not(even))
            def _():
                @pl.when(t + 1 < NCH)
                def _():
                    pltpu.async_copy(hw2.at[src_v.at[pl.ds((t + 1) * K, K)]],
                                     rows_a, gsem)
                pltpu.sync_copy(rows_b, acc.at[loc_v.at[t, 0]], add=True)
            return 0

        lax.fori_loop(0, NCH, body, 0)
        plsc.subcore_barrier()

        pltpu.sync_copy(acc.at[pl.ds(s * OCH, OCH)],
                        out.at[p, pl.ds(base + s * OCH, OCH)])
        plsc.subcore_barrier()


@functools.partial(
    pl.kernel,
    mesh=_mesh,
    compiler_params=pltpu.CompilerParams(use_tc_tiling_on_sc=False),
    out_type=jax.ShapeDtypeStruct((TN, 16), jnp.float32),
    scratch_types=[
        pltpu.VMEM((EPS,), jnp.int32),
        pltpu.VMEM((NCH, 1, K), jnp.int32),
        pltpu.VMEM((K, 16), jnp.float32),
        pltpu.VMEM_SHARED((PAD_ROWS, 16), jnp.float32),
    ],
)
def _sc_cnt(dst, zeros, out, dst_v, loc_v, ones_v, acc):
    c = lax.axis_index("c")
    s = lax.axis_index("s")
    base = c * HALF

    pltpu.sync_copy(zeros.at[pl.ds(s * ZCH, ZCH)], acc.at[pl.ds(s * ZCH, ZCH)])

    e0 = s * EPS
    pltpu.sync_copy(dst.at[pl.ds(e0, EPS)], dst_v)
    _compute_local_dst(dst_v, loc_v, base)

    def fill(i, _):
        ones_v[i] = jnp.full((16,), 1.0, jnp.float32)
        return 0
    lax.fori_loop(0, K, fill, 0)
    plsc.subcore_barrier()

    def body(t, _):
        pltpu.sync_copy(ones_v, acc.at[loc_v.at[t, 0]], add=True)
        return 0
    lax.fori_loop(0, NCH, body, 0)
    plsc.subcore_barrier()

    pltpu.sync_copy(acc.at[pl.ds(s * OCH, OCH)],
                    out.at[pl.ds(base + s * OCH, OCH)])


def _silu(x):
    return x * jax.nn.sigmoid(x)


def _proj_body(x_ref, w_ref, b_ref, g_ref, lb_ref, o_ref):
    y = jnp.dot(x_ref[...], w_ref[...], preferred_element_type=jnp.float32)
    y = y + b_ref[...]
    m = jnp.mean(y, axis=-1, keepdims=True)
    v = jnp.mean((y - m) ** 2, axis=-1, keepdims=True)
    y = (y - m) * lax.rsqrt(v + 1e-5) * g_ref[...] + lb_ref[...]
    o_ref[...] = _silu(y)


def _proj(x, w, b, g, lb, br=1000):
    nb = N // br
    vec = pl.BlockSpec((1, H), lambda i: (0, 0))
    return pl.pallas_call(
        _proj_body,
        grid=(nb,),
        in_specs=[
            pl.BlockSpec((br, D), lambda i: (i, 0)),
            pl.BlockSpec((D, H), lambda i: (0, 0)),
            vec, vec, vec,
        ],
        out_specs=pl.BlockSpec((br, H), lambda i: (i, 0)),
        out_shape=jax.ShapeDtypeStruct((N, H), jnp.float32),
    )(x, w, b, g, lb)


def _relmm_body(h_ref, wr_ref, wroot_ref, cb_ref, root_ref, hw_ref):
    h = h_ref[...]
    root_ref[...] = (
        jnp.dot(h, wroot_ref[...], preferred_element_type=jnp.float32)
        + cb_ref[...]
    )
    for r in range(R):
        hw_ref[r] = jnp.dot(h, wr_ref[r], preferred_element_type=jnp.float32)


def _relmm(h, w_rel_l, w_root_l, conv_b_l, br=1000):
    nb = TN // br
    return pl.pallas_call(
        _relmm_body,
        grid=(nb,),
        in_specs=[
            pl.BlockSpec((br, H), lambda i: (i, 0)),
            pl.BlockSpec((R, H, H), lambda i: (0, 0, 0)),
            pl.BlockSpec((H, H), lambda i: (0, 0)),
            pl.BlockSpec((1, H), lambda i: (0, 0)),
        ],
        out_specs=[
            pl.BlockSpec((br, H), lambda i: (i, 0)),
            pl.BlockSpec((R, br, H), lambda i: (0, i, 0)),
        ],
        out_shape=[
            jax.ShapeDtypeStruct((TN, H), jnp.float32),
            jax.ShapeDtypeStruct((R, TN, H), jnp.float32),
        ],
    )(h, w_rel_l, w_root_l, conv_b_l)


def _comb_body(root_ref, a0_ref, a1_ref, a2_ref, c0_ref, c1_ref, c2_ref,
               out_ref, st_ref):
    o = root_ref[...]
    for a_ref, c_ref in ((a0_ref, c0_ref), (a1_ref, c1_ref), (a2_ref, c2_ref)):
        inv = 1.0 / jnp.maximum(c_ref[...][:, 0:1], 1.0)
        o = o + a_ref[...] * inv
    out_ref[...] = o
    i = pl.program_id(0)

    @pl.when(i == 0)
    def _():
        st_ref[...] = jnp.zeros_like(st_ref)
    st_ref[0:1, :] += jnp.sum(o, axis=0, keepdims=True)
    st_ref[1:2, :] += jnp.sum(o * o, axis=0, keepdims=True)


def _comb(root, aggs, cnts, br=1000):
    nb = TN // br
    blk = pl.BlockSpec((br, H), lambda i: (i, 0))
    cblk = pl.BlockSpec((br, 16), lambda i: (i, 0))
    return pl.pallas_call(
        _comb_body,
        grid=(nb,),
        in_specs=[blk, blk, blk, blk, cblk, cblk, cblk],
        out_specs=[blk, pl.BlockSpec((8, H), lambda i: (0, 0))],
        out_shape=[
            jax.ShapeDtypeStruct((TN, H), jnp.float32),
            jax.ShapeDtypeStruct((8, H), jnp.float32),
        ],
    )(root, aggs[0], aggs[1], aggs[2], cnts[0], cnts[1], cnts[2])


def _bn_body(o_ref, st_ref, hin_ref, g_ref, b_ref, out_ref):
    m = st_ref[0:1, :] * (1.0 / TN)
    ex2 = st_ref[1:2, :] * (1.0 / TN)
    v = ex2 - m * m
    y = (o_ref[...] - m) * lax.rsqrt(v + 1e-5) * g_ref[...] + b_ref[...]
    out_ref[...] = _silu(y) + hin_ref[...]


def _bn(o, st, h_in, g_l, b_l, br=1000):
    nb = TN // br
    blk = pl.BlockSpec((br, H), lambda i: (i, 0))
    return pl.pallas_call(
        _bn_body,
        grid=(nb,),
        in_specs=[
            blk,
            pl.BlockSpec((8, H), lambda i: (0, 0)),
            blk,
            pl.BlockSpec((1, H), lambda i: (0, 0)),
            pl.BlockSpec((1, H), lambda i: (0, 0)),
        ],
        out_specs=blk,
        out_shape=jax.ShapeDtypeStruct((TN, H), jnp.float32),
    )(o, st, h_in, g_l, b_l)


def kernel(x_drug, x_protein, edge_index_dd, edge_index_dp, edge_index_pp,
           proj_W, proj_b, ln_g, ln_b, W_rel, W_root, conv_b, bn_g, bn_b):
    hd = _proj(x_drug, proj_W[0], proj_b[0:1], ln_g[0:1], ln_b[0:1])
    hp = _proj(x_protein, proj_W[1], proj_b[1:2], ln_g[1:2], ln_b[1:2])
    h = jnp.concatenate([hd, hp], axis=0)

    i32 = jnp.int32
    srcs = [
        edge_index_dd[0].astype(i32),
        edge_index_dp[0].astype(i32),
        (edge_index_pp[0] + N).astype(i32),
    ]
    dsts = [
        edge_index_dd[1].astype(i32),
        (edge_index_dp[1] + N).astype(i32),
        (edge_index_pp[1] + N).astype(i32),
    ]
    srcs0 = [s * 2 for s in srcs]
    srcs1 = [s * 2 + 1 for s in srcs]

    zeros_hh = jnp.zeros((PAD_ROWS, HH), jnp.float32)
    zeros_16 = jnp.zeros((PAD_ROWS, 16), jnp.float32)
    cnts = [_sc_cnt(dsts[r], zeros_16) for r in range(R)]

    for l in range(L):
        h_in = h
        root, hw = _relmm(h, W_rel[l], W_root[l], conv_b[l:l + 1])
        aggs = []
        for r in range(R):
            halves = _sc_agg(hw[r].reshape(2 * TN, HH), dsts[r],
                             srcs0[r], srcs1[r], zeros_hh)
            aggs.append(jnp.concatenate([halves[0], halves[1]], axis=1))
        o, st = _comb(root, aggs, cnts)
        h = _bn(o, st, h_in, bn_g[l:l + 1], bn_b[l:l + 1])

    return (h[:N], h[N:])
